# baseline (speedup 1.0000x reference)
import functools

import jax
import jax.numpy as jnp
from jax import lax
from jax.experimental import pallas as pl
from jax.experimental.pallas import tpu as pltpu
from jax.experimental.pallas import tpu_sc as plsc

NUM_CORES = 2
NUM_SUBCORES = 16
LANES = 16
CHUNK = 8
NSLOT = 3


def _build_sc_kernel(B, N, T, D):
    n_workers = NUM_CORES * NUM_SUBCORES
    t_w = T // n_workers
    n_chunks = t_w // CHUNK
    n_main = (n_chunks - 2) // NSLOT * NSLOT
    vecs_per_row = D // LANES

    mesh = plsc.VectorSubcoreMesh(
        core_axis_name="c", subcore_axis_name="s",
        num_cores=NUM_CORES, num_subcores=NUM_SUBCORES)

    @functools.partial(
        pl.kernel,
        out_type=jax.ShapeDtypeStruct((N, D), jnp.float32),
        mesh=mesh,
        scratch_types=[
            pltpu.VMEM((B * t_w,), jnp.int32),
            pltpu.VMEM((NSLOT, B, CHUNK, D), jnp.float32),
            pltpu.VMEM((NSLOT, CHUNK, D), jnp.float32),
            [pltpu.SemaphoreType.DMA] * NSLOT,
            [pltpu.SemaphoreType.DMA] * NSLOT,
            [pltpu.SemaphoreType.DMA] * NSLOT,
        ],
    )
    def sc_kernel(x_hbm, tok_hbm, pos_hbm, out_hbm, idx_v, gbuf, pbuf,
                  gsems, psems, osems):
        wid = lax.axis_index("s") * NUM_CORES + lax.axis_index("c")
        base_t = wid * t_w

        for b in range(B):
            pltpu.sync_copy(x_hbm.at[pl.ds(b * T + base_t, t_w)],
                            idx_v.at[pl.ds(b * t_w, t_w)])

        def fire_in(j, slot):
            t_off = j * CHUNK
            pltpu.async_copy(
                pos_hbm.at[pl.ds(base_t + t_off, CHUNK)], pbuf.at[slot],
                psems[slot])
            for b in range(B):
                pltpu.async_copy(
                    tok_hbm.at[idx_v.at[pl.ds(b * t_w + t_off, CHUNK)]],
                    gbuf.at[slot, b], gsems[slot])

        def wait_in(j, slot):
            t_off = j * CHUNK
            pltpu.make_async_copy(
                pos_hbm.at[pl.ds(base_t + t_off, CHUNK)], pbuf.at[slot],
                psems[slot]).wait()
            for b in range(B):
                pltpu.make_async_copy(
                    tok_hbm.at[idx_v.at[pl.ds(b * t_w + t_off, CHUNK)]],
                    gbuf.at[slot, b], gsems[slot]).wait()

        def fire_out(j, slot):
            t_off = j * CHUNK
            for b in range(B):
                pltpu.async_copy(
                    gbuf.at[slot, b],
                    out_hbm.at[pl.ds(b * T + base_t + t_off, CHUNK)],
                    osems[slot])

        def wait_out(j, slot):
            t_off = j * CHUNK
            for b in range(B):
                pltpu.make_async_copy(
                    gbuf.at[slot, b],
                    out_hbm.at[pl.ds(b * T + base_t + t_off, CHUNK)],
                    osems[slot]).wait()

        def add_chunk(sl):
            def add_row(r, c):
                for col in range(vecs_per_row):
                    vsl = pl.ds(col * LANES, LANES)
                    vp = pbuf[sl, r, vsl]
                    for b in range(B):
                        gbuf[sl, b, r, vsl] = gbuf[sl, b, r, vsl] + vp
                return c
            lax.fori_loop(0, CHUNK, add_row, 0)

        def body(j, sl, prefetch):
            nxt = (sl + 1) % NSLOT
            if prefetch:
                @pl.when(j >= 2)
                def _():
                    wait_out(j - 2, nxt)
                fire_in(j + 1, nxt)
            wait_in(j, sl)
            add_chunk(sl)
            fire_out(j, sl)

        fire_in(0, 0)

        @pl.loop(0, n_main, step=NSLOT)
        def pipeline(jj):
            for k in range(NSLOT):
                body(jj + k, k, True)

        for j in range(n_main, n_chunks):
            body(j, j % NSLOT, j + 1 < n_chunks)

        for j in range(n_chunks - NSLOT, n_chunks):
            wait_out(j, j % NSLOT)

    return sc_kernel


def kernel(x, tok_emb, pos_emb):
    B, T = x.shape
    V, D = tok_emb.shape
    N = B * T
    sc_kernel = _build_sc_kernel(B, N, T, D)
    out = sc_kernel(x.reshape(N), tok_emb, pos_emb)
    return out.reshape(B, T, D)
